# baseline (speedup 1.0000x reference)
import functools
import jax
import jax.numpy as jnp
from jax import lax
from jax.experimental import pallas as pl
from jax.experimental.pallas import tpu as pltpu
from jax.experimental.pallas import tpu_sc as plsc

NC = 2
NS = 16
L = 16
NW = NC * NS


def _make_kernel(B, V, F):
    assert B % (NW * L) == 0 and F % L == 0
    b_per_w = B // NW
    CH = 128
    n_ch = b_per_w // CH
    idx_rows = B // CH
    mesh = plsc.VectorSubcoreMesh(
        core_axis_name="c", subcore_axis_name="s", num_cores=NC, num_subcores=NS
    )

    @functools.partial(
        pl.kernel,
        out_type=jax.ShapeDtypeStruct((B,), jnp.float32),
        mesh=mesh,
        compiler_params=pltpu.CompilerParams(
            needs_layout_passes=False, use_tc_tiling_on_sc=False
        ),
        scratch_types=[
            pltpu.VMEM((n_ch, CH), jnp.int32),
            pltpu.VMEM((n_ch, CH), jnp.int32),
            pltpu.VMEM((b_per_w, F), jnp.float32),
            pltpu.VMEM((b_per_w, F), jnp.float32),
            pltpu.VMEM((b_per_w,), jnp.float32),
            pltpu.SemaphoreType.DMA,
        ],
    )
    def k(n1_hbm, n2_hbm, tab_hbm, out_hbm, idx1_v, idx2_v, rows1_v, rows2_v, out_v, sem):
        wid = lax.axis_index("s") * NC + lax.axis_index("c")
        base = wid * b_per_w
        crow = wid * n_ch

        d1 = pltpu.async_copy(n1_hbm.at[pl.ds(crow, n_ch)], idx1_v, sem)
        d2 = pltpu.async_copy(n2_hbm.at[pl.ds(crow, n_ch)], idx2_v, sem)
        d1.wait()
        d2.wait()
        descs = []
        for j in range(n_ch):
            descs.append(
                pltpu.async_copy(
                    tab_hbm.at[idx1_v.at[j]], rows1_v.at[pl.ds(j * CH, CH)], sem
                )
            )
            descs.append(
                pltpu.async_copy(
                    tab_hbm.at[idx2_v.at[j]], rows2_v.at[pl.ds(j * CH, CH)], sem
                )
            )
        for d in descs:
            d.wait()

        lane = lax.iota(jnp.int32, 16)

        def group(g, carry):
            row = lane + g * L
            acc = jnp.zeros((L,), jnp.float32)
            for f in range(F):
                col = jnp.full((L,), f, jnp.int32)
                a = plsc.load_gather(rows1_v, [row, col])
                b = plsc.load_gather(rows2_v, [row, col])
                acc = acc + a * b
            out_v[pl.ds(g * L, L)] = acc
            return carry

        lax.fori_loop(0, b_per_w // L, group, 0)
        pltpu.sync_copy(out_v, out_hbm.at[pl.ds(base, b_per_w)])

    return k


@jax.jit
def kernel(node1, node2, node_factors):
    B = node1.shape[0]
    V, F = node_factors.shape
    CH = 128
    n1 = node1.reshape(B // CH, CH)
    n2 = node2.reshape(B // CH, CH)
    k = _make_kernel(B, V, F)
    return k(n1, n2, node_factors)
